# baseline (speedup 1.0000x reference)
import functools

import jax
import jax.numpy as jnp
from jax import lax
from jax.experimental import pallas as pl
from jax.experimental.pallas import tpu as pltpu
from jax.experimental.pallas import tpu_sc as plsc

N_FIELDS = 26
VOCAB = 100000
WIDTH = 32
BATCH = 16384

PACK = 4
NTILE = 782
ROWS4_F = NTILE * 32
ROWS4 = N_FIELDS * ROWS4_F

NC = 2
NS = 16
NW = NC * NS
JPW = 25
BPW = BATCH // NW
BLK = 128
NBLK = BPW // BLK
G16 = 16
NG = BLK // G16
LANES = 16

_mesh = plsc.VectorSubcoreMesh(core_axis_name="c", subcore_axis_name="s")


@functools.partial(
    pl.kernel,
    out_type=jax.ShapeDtypeStruct((ROWS4, 128), jnp.float32),
    mesh=_mesh,
    scratch_types=[
        pltpu.VMEM((PACK, 8, 128), jnp.float32),
        pltpu.VMEM((PACK, 8, 128), jnp.float32),
        pltpu.SemaphoreType.DMA,
        pltpu.SemaphoreType.DMA,
    ],
    compiler_params=pltpu.CompilerParams(needs_layout_passes=False),
)
def _detile(tab_hbm, out_hbm, stage, packbuf, sem_i, sem_o):
    wid = lax.axis_index("s") * NC + lax.axis_index("c")
    j0 = wid * JPW
    nj = lax.min(NTILE - j0, JPW)
    lane = lax.iota(jnp.int32, LANES)

    i0s, i1s, quads = [], [], []
    for cc in range(8):
        c16 = cc * LANES + lane
        w = c16 & 31
        i0s.append(w >> 3)
        i1s.append(w & 7)
        quads.append(c16 >> 5)

    def per_field(f, carry):
        def per_j(jj, c):
            j = j0 + jj
            for w8 in range(PACK):
                pltpu.async_copy(
                    tab_hbm.at[f, pl.ds(w8 * 8, 8), pl.ds(j * 128, 128)],
                    stage.at[w8],
                    sem_i,
                )
            for w8 in range(PACK):
                pltpu.make_async_copy(
                    tab_hbm.at[f, pl.ds(w8 * 8, 8), pl.ds(j * 128, 128)],
                    stage.at[w8],
                    sem_i,
                ).wait()

            for cc in range(8):
                for r in range(32):
                    val = plsc.load_gather(stage, [i0s[cc], i1s[cc], quads[cc] + r * PACK])
                    packbuf[r >> 3, r & 7, pl.ds(cc * LANES, LANES)] = val

            r0 = f * ROWS4_F + j * 32
            for t in range(PACK):
                pltpu.async_copy(
                    packbuf.at[t],
                    out_hbm.at[pl.ds(r0 + t * 8, 8), :],
                    sem_o,
                )
            for t in range(PACK):
                pltpu.make_async_copy(
                    packbuf.at[t],
                    out_hbm.at[pl.ds(r0 + t * 8, 8), :],
                    sem_o,
                ).wait()
            return c

        lax.fori_loop(0, nj, per_j, 0)
        return carry

    lax.fori_loop(0, N_FIELDS, per_field, 0)


@functools.partial(
    pl.kernel,
    out_type=jax.ShapeDtypeStruct((N_FIELDS, WIDTH, BATCH), jnp.float32),
    mesh=_mesh,
    scratch_types=[
        pltpu.VMEM((BPW,), jnp.int32),
        pltpu.VMEM((BPW,), jnp.int32),
        pltpu.VMEM((BPW,), jnp.int32),
        pltpu.VMEM((BLK, 128), jnp.float32),
        pltpu.VMEM((PACK, 8, BLK), jnp.float32),
        pltpu.SemaphoreType.DMA,
        pltpu.SemaphoreType.DMA,
    ],
    compiler_params=pltpu.CompilerParams(needs_layout_passes=False),
)
def _embed_gather(xs_hbm, tab4_hbm, out_hbm, xsv, idx4, subv, rows, tiles,
                  sem_g, sem_o):
    wid = lax.axis_index("s") * NC + lax.axis_index("c")
    base_b = wid * BPW
    lane = lax.iota(jnp.int32, LANES)

    def per_field(f, carry):
        pltpu.sync_copy(xs_hbm.at[pl.ds(f * BATCH + base_b, BPW)], xsv)
        fbase = f * ROWS4_F

        def prep(i, c):
            sl = pl.ds(i * LANES, LANES)
            v = xsv[sl]
            idx4[sl] = fbase + (v >> 2)
            subv[sl] = (v & 3) * WIDTH
            return c

        lax.fori_loop(0, BPW // LANES, prep, 0)

        def per_block(blk, c):
            def fire(g, c2):
                pltpu.async_copy(
                    tab4_hbm.at[idx4.at[pl.ds(blk * BLK + g * G16, G16)]],
                    rows.at[pl.ds(g * G16, G16)],
                    sem_g,
                )
                return c2

            lax.fori_loop(0, NG, fire, 0)

            def drain(g, c2):
                pltpu.make_async_copy(
                    tab4_hbm.at[idx4.at[pl.ds(blk * BLK + g * G16, G16)]],
                    rows.at[pl.ds(g * G16, G16)],
                    sem_g,
                ).wait()
                return c2

            lax.fori_loop(0, NG, drain, 0)

            def asm(kk, c2):
                b16 = kk * LANES + lane
                sub16 = subv[pl.ds(blk * BLK + kk * LANES, LANES)]
                for w in range(WIDTH):
                    val = plsc.load_gather(rows, [b16, sub16 + w])
                    tiles[w // 8, w % 8, pl.ds(kk * LANES, LANES)] = val
                return c2

            lax.fori_loop(0, BLK // LANES, asm, 0)

            b0 = base_b + blk * BLK
            for w8 in range(PACK):
                pltpu.async_copy(
                    tiles.at[w8],
                    out_hbm.at[f, pl.ds(w8 * 8, 8), pl.ds(b0, BLK)],
                    sem_o,
                )
            for w8 in range(PACK):
                pltpu.make_async_copy(
                    tiles.at[w8],
                    out_hbm.at[f, pl.ds(w8 * 8, 8), pl.ds(b0, BLK)],
                    sem_o,
                ).wait()
            return c

        lax.fori_loop(0, NBLK, per_block, 0)
        return carry

    lax.fori_loop(0, N_FIELDS, per_field, 0)


def kernel(xs, tables):
    xs_flat = xs.reshape(N_FIELDS * BATCH)
    tab_t = jnp.transpose(tables, (0, 2, 1))
    tab4 = _detile(tab_t)
    out_t = _embed_gather(xs_flat, tab4)
    return jnp.transpose(out_t, (0, 2, 1))


# speedup vs baseline: 1.3248x; 1.3248x over previous
import functools

import jax
import jax.numpy as jnp
from jax import lax
from jax.experimental import pallas as pl
from jax.experimental.pallas import tpu as pltpu
from jax.experimental.pallas import tpu_sc as plsc

N_FIELDS = 26
VOCAB = 100000
WIDTH = 32
BATCH = 16384

PACK = 4
NTILE = 782
ROWS4_F = NTILE * 32
ROWS4 = N_FIELDS * ROWS4_F

NC = 2
NS = 16
NW = NC * NS
JPW = 25
JMAC = 5
MPF = JPW // JMAC
MTOT_A = N_FIELDS * MPF

BPW = BATCH // NW
BLK = 128
NBLK = BPW // BLK
MTOT_B = N_FIELDS * NBLK
LANES = 16

_mesh = plsc.VectorSubcoreMesh(core_axis_name="c", subcore_axis_name="s")


@functools.partial(
    pl.kernel,
    out_type=jax.ShapeDtypeStruct((ROWS4, 128), jnp.float32),
    mesh=_mesh,
    scratch_types=[
        pltpu.VMEM((2, PACK, JMAC, 8, 128), jnp.float32),
        pltpu.VMEM((2, PACK * JMAC, 8, 128), jnp.float32),
        pltpu.SemaphoreType.DMA,
        pltpu.SemaphoreType.DMA,
        pltpu.SemaphoreType.DMA,
        pltpu.SemaphoreType.DMA,
    ],
    compiler_params=pltpu.CompilerParams(needs_layout_passes=False),
)
def _detile(tab_hbm, out_hbm, stage, packbuf, si0, si1, so0, so1):
    wid = lax.axis_index("s") * NC + lax.axis_index("c")
    j0 = lax.min(wid * JPW, NTILE - JPW)
    lane = lax.iota(jnp.int32, LANES)

    i0s, i1s, quads = [], [], []
    for cc in range(8):
        c16 = cc * LANES + lane
        w = c16 & 31
        i0s.append(w >> 3)
        i1s.append(w & 7)
        quads.append(c16 >> 5)

    def step_fm(mg):
        return mg // MPF, mg % MPF

    def fire_in(mg, slot, sem):
        f, m = step_fm(mg)
        jbase = (j0 + m * JMAC) * 128

        def go(q, c):
            w8 = q & 3
            jp = q >> 2
            pltpu.async_copy(
                tab_hbm.at[f, pl.ds(w8 * 8, 8), pl.ds(jbase + jp * 128, 128)],
                stage.at[slot, w8, jp],
                sem,
            )
            return c

        lax.fori_loop(0, PACK * JMAC, go, 0)

    def drain_in(mg, slot, sem):
        f, m = step_fm(mg)
        jbase = (j0 + m * JMAC) * 128

        def go(q, c):
            w8 = q & 3
            jp = q >> 2
            pltpu.make_async_copy(
                tab_hbm.at[f, pl.ds(w8 * 8, 8), pl.ds(jbase + jp * 128, 128)],
                stage.at[slot, w8, jp],
                sem,
            ).wait()
            return c

        lax.fori_loop(0, PACK * JMAC, go, 0)

    def fire_out(mg, slot, sem):
        f, m = step_fm(mg)
        r0 = f * ROWS4_F + (j0 + m * JMAC) * 32

        def go(q, c):
            pltpu.async_copy(
                packbuf.at[slot, q],
                out_hbm.at[pl.ds(r0 + q * 8, 8), :],
                sem,
            )
            return c

        lax.fori_loop(0, PACK * JMAC, go, 0)

    def drain_out(mg, slot, sem):
        f, m = step_fm(mg)
        r0 = f * ROWS4_F + (j0 + m * JMAC) * 32

        def go(q, c):
            pltpu.make_async_copy(
                packbuf.at[slot, q],
                out_hbm.at[pl.ds(r0 + q * 8, 8), :],
                sem,
            ).wait()
            return c

        lax.fori_loop(0, PACK * JMAC, go, 0)

    def by_par(slot, fn0, fn1):
        @pl.when(slot == 0)
        def _():
            fn0()

        @pl.when(slot != 0)
        def _():
            fn1()

    fire_in(0, 0, si0)

    def body(mg, carry):
        slot = mg & 1
        sslot = lane * 0 + slot

        @pl.when(mg + 1 < MTOT_A)
        def _():
            by_par(1 - slot,
                   lambda: fire_in(mg + 1, 1 - slot, si0),
                   lambda: fire_in(mg + 1, 1 - slot, si1))

        by_par(slot,
               lambda: drain_in(mg, slot, si0),
               lambda: drain_in(mg, slot, si1))

        @pl.when(mg >= 2)
        def _():
            by_par(slot,
                   lambda: drain_out(mg - 2, slot, so0),
                   lambda: drain_out(mg - 2, slot, so1))

        def asm(jp, c):
            sjp = lane * 0 + jp
            for cc in range(8):
                for r in range(32):
                    val = plsc.load_gather(
                        stage, [sslot, i0s[cc], sjp, i1s[cc], quads[cc] + r * PACK]
                    )
                    rt = jp * 32 + r
                    packbuf[slot, jp * PACK + (r >> 3), r & 7,
                            pl.ds(cc * LANES, LANES)] = val
                    del rt
            return c

        lax.fori_loop(0, JMAC, asm, 0)

        by_par(slot,
               lambda: fire_out(mg, slot, so0),
               lambda: fire_out(mg, slot, so1))
        return carry

    lax.fori_loop(0, MTOT_A, body, 0)
    drain_out(MTOT_A - 2, (MTOT_A - 2) & 1, so0)
    drain_out(MTOT_A - 1, (MTOT_A - 1) & 1, so1)


@functools.partial(
    pl.kernel,
    out_type=jax.ShapeDtypeStruct((N_FIELDS, WIDTH, BATCH), jnp.float32),
    mesh=_mesh,
    scratch_types=[
        pltpu.VMEM((BPW,), jnp.int32),
        pltpu.VMEM((2, BPW), jnp.int32),
        pltpu.VMEM((2, BPW), jnp.int32),
        pltpu.VMEM((2, BLK // 8, 8, 128), jnp.float32),
        pltpu.VMEM((2, PACK, 8, BLK), jnp.float32),
        pltpu.SemaphoreType.DMA,
        pltpu.SemaphoreType.DMA,
        pltpu.SemaphoreType.DMA,
        pltpu.SemaphoreType.DMA,
    ],
    compiler_params=pltpu.CompilerParams(needs_layout_passes=False),
)
def _embed_gather(xs_hbm, tab4_hbm, out_hbm, xsv, idx4, subv, rows, tiles,
                  sg0, sg1, so0, so1):
    wid = lax.axis_index("s") * NC + lax.axis_index("c")
    base_b = wid * BPW
    lane = lax.iota(jnp.int32, LANES)

    def prep_field(f):
        pltpu.sync_copy(xs_hbm.at[pl.ds(f * BATCH + base_b, BPW)], xsv)
        fslot = f & 1
        fbase = f * ROWS4_F

        def go(i, c):
            sl = pl.ds(i * LANES, LANES)
            v = xsv[sl]
            idx4[fslot, sl] = fbase + (v >> 2)
            subv[fslot, sl] = (v & 3) * WIDTH
            return c

        lax.fori_loop(0, BPW // LANES, go, 0)

    def fire_g(mg, slot, sem):
        f = mg >> 2
        blk = mg & 3
        fslot = f & 1

        def go(g, c):
            pltpu.async_copy(
                tab4_hbm.at[idx4.at[fslot, pl.ds(blk * BLK + g * 8, 8)]],
                rows.at[slot, g],
                sem,
            )
            return c

        lax.fori_loop(0, BLK // 8, go, 0)

    def drain_g(mg, slot, sem):
        f = mg >> 2
        blk = mg & 3
        fslot = f & 1

        def go(g, c):
            pltpu.make_async_copy(
                tab4_hbm.at[idx4.at[fslot, pl.ds(blk * BLK + g * 8, 8)]],
                rows.at[slot, g],
                sem,
            ).wait()
            return c

        lax.fori_loop(0, BLK // 8, go, 0)

    def fire_out(mg, slot, sem):
        f = mg >> 2
        blk = mg & 3
        b0 = base_b + blk * BLK

        def go(w8, c):
            pltpu.async_copy(
                tiles.at[slot, w8],
                out_hbm.at[f, pl.ds(w8 * 8, 8), pl.ds(b0, BLK)],
                sem,
            )
            return c

        lax.fori_loop(0, PACK, go, 0)

    def drain_out(mg, slot, sem):
        f = mg >> 2
        blk = mg & 3
        b0 = base_b + blk * BLK

        def go(w8, c):
            pltpu.make_async_copy(
                tiles.at[slot, w8],
                out_hbm.at[f, pl.ds(w8 * 8, 8), pl.ds(b0, BLK)],
                sem,
            ).wait()
            return c

        lax.fori_loop(0, PACK, go, 0)

    def by_par(slot, fn0, fn1):
        @pl.when(slot == 0)
        def _():
            fn0()

        @pl.when(slot != 0)
        def _():
            fn1()

    prep_field(0)
    fire_g(0, 0, sg0)

    def body(mg, carry):
        slot = mg & 1
        blk = mg & 3
        sslot = lane * 0 + slot

        @pl.when(mg + 1 < MTOT_B)
        def _():
            @pl.when(blk == 3)
            def _():
                prep_field((mg >> 2) + 1)

            by_par(1 - slot,
                   lambda: fire_g(mg + 1, 1 - slot, sg0),
                   lambda: fire_g(mg + 1, 1 - slot, sg1))

        by_par(slot,
               lambda: drain_g(mg, slot, sg0),
               lambda: drain_g(mg, slot, sg1))

        @pl.when(mg >= 2)
        def _():
            by_par(slot,
                   lambda: drain_out(mg - 2, slot, so0),
                   lambda: drain_out(mg - 2, slot, so1))

        fslot = (mg >> 2) & 1

        def asm(kk, c):
            b16 = kk * LANES + lane
            sub16 = subv[fslot, pl.ds(blk * BLK + kk * LANES, LANES)]
            for w in range(WIDTH):
                val = plsc.load_gather(
                    rows, [sslot, b16 >> 3, b16 & 7, sub16 + w]
                )
                tiles[slot, w // 8, w % 8, pl.ds(kk * LANES, LANES)] = val
            return c

        lax.fori_loop(0, BLK // LANES, asm, 0)

        by_par(slot,
               lambda: fire_out(mg, slot, so0),
               lambda: fire_out(mg, slot, so1))
        return carry

    lax.fori_loop(0, MTOT_B, body, 0)
    drain_out(MTOT_B - 2, (MTOT_B - 2) & 1, so0)
    drain_out(MTOT_B - 1, (MTOT_B - 1) & 1, so1)


def kernel(xs, tables):
    xs_flat = xs.reshape(N_FIELDS * BATCH)
    tab_t = jnp.transpose(tables, (0, 2, 1))
    tab4 = _detile(tab_t)
    out_t = _embed_gather(xs_flat, tab4)
    return jnp.transpose(out_t, (0, 2, 1))


# speedup vs baseline: 2.7273x; 2.0587x over previous
import functools

import jax
import jax.numpy as jnp
from jax import lax
from jax.experimental import pallas as pl
from jax.experimental.pallas import tpu as pltpu
from jax.experimental.pallas import tpu_sc as plsc

N_FIELDS = 26
VOCAB = 100000
WIDTH = 32
BATCH = 16384

PACK = 4
NTILE = 782
ROWS4_F = NTILE * 32
ROWS4 = N_FIELDS * ROWS4_F

NC = 2
NS = 16
NW = NC * NS
JPW = 25
JMAC = 5
MPF = JPW // JMAC
MTOT_A = N_FIELDS * MPF

BPW = BATCH // NW
BLK = 128
NBLK = BPW // BLK
MTOT_B = N_FIELDS * NBLK
LANES = 16

_mesh = plsc.VectorSubcoreMesh(core_axis_name="c", subcore_axis_name="s")


@functools.partial(
    pl.kernel,
    out_type=jax.ShapeDtypeStruct((ROWS4, 128), jnp.float32),
    mesh=_mesh,
    scratch_types=[
        pltpu.VMEM((2, PACK, JMAC, 8, 128), jnp.float32),
        pltpu.VMEM((2, PACK * JMAC, 8, 128), jnp.float32),
        pltpu.SemaphoreType.DMA,
        pltpu.SemaphoreType.DMA,
        pltpu.SemaphoreType.DMA,
        pltpu.SemaphoreType.DMA,
    ],
    compiler_params=pltpu.CompilerParams(needs_layout_passes=False),
)
def _detile(tab_hbm, out_hbm, stage, packbuf, si0, si1, so0, so1):
    wid = lax.axis_index("s") * NC + lax.axis_index("c")
    j0 = lax.min(wid * JPW, NTILE - JPW)
    lane = lax.iota(jnp.int32, LANES)

    i0s, i1s, quads = [], [], []
    for cc in range(8):
        c16 = cc * LANES + lane
        i0s.append(c16 >> 5)
        i1s.append((c16 >> 2) & 7)
        quads.append(c16 & 3)

    def step_fm(mg):
        return mg // MPF, mg % MPF

    def fire_in(mg, slot, sem):
        f, m = step_fm(mg)
        jbase = (j0 + m * JMAC) * 128

        def go(q, c):
            w8 = q & 3
            jp = q >> 2
            pltpu.async_copy(
                tab_hbm.at[f, pl.ds(w8 * 8, 8), pl.ds(jbase + jp * 128, 128)],
                stage.at[slot, w8, jp],
                sem,
            )
            return c

        lax.fori_loop(0, PACK * JMAC, go, 0)

    def drain_in(mg, slot, sem):
        f, m = step_fm(mg)
        jbase = (j0 + m * JMAC) * 128

        def go(q, c):
            w8 = q & 3
            jp = q >> 2
            pltpu.make_async_copy(
                tab_hbm.at[f, pl.ds(w8 * 8, 8), pl.ds(jbase + jp * 128, 128)],
                stage.at[slot, w8, jp],
                sem,
            ).wait()
            return c

        lax.fori_loop(0, PACK * JMAC, go, 0)

    def fire_out(mg, slot, sem):
        f, m = step_fm(mg)
        r0 = f * ROWS4_F + (j0 + m * JMAC) * 32

        def go(q, c):
            pltpu.async_copy(
                packbuf.at[slot, q],
                out_hbm.at[pl.ds(r0 + q * 8, 8), :],
                sem,
            )
            return c

        lax.fori_loop(0, PACK * JMAC, go, 0)

    def drain_out(mg, slot, sem):
        f, m = step_fm(mg)
        r0 = f * ROWS4_F + (j0 + m * JMAC) * 32

        def go(q, c):
            pltpu.make_async_copy(
                packbuf.at[slot, q],
                out_hbm.at[pl.ds(r0 + q * 8, 8), :],
                sem,
            ).wait()
            return c

        lax.fori_loop(0, PACK * JMAC, go, 0)

    def by_par(slot, fn0, fn1):
        @pl.when(slot == 0)
        def _():
            fn0()

        @pl.when(slot != 0)
        def _():
            fn1()

    fire_in(0, 0, si0)

    def body(mg, carry):
        slot = mg & 1
        sslot = lane * 0 + slot

        @pl.when(mg + 1 < MTOT_A)
        def _():
            by_par(1 - slot,
                   lambda: fire_in(mg + 1, 1 - slot, si0),
                   lambda: fire_in(mg + 1, 1 - slot, si1))

        by_par(slot,
               lambda: drain_in(mg, slot, si0),
               lambda: drain_in(mg, slot, si1))

        @pl.when(mg >= 2)
        def _():
            by_par(slot,
                   lambda: drain_out(mg - 2, slot, so0),
                   lambda: drain_out(mg - 2, slot, so1))

        def asm(jp, c):
            sjp = lane * 0 + jp
            for cc in range(8):
                for r in range(32):
                    val = plsc.load_gather(
                        stage, [sslot, i0s[cc], sjp, i1s[cc], quads[cc] + r * PACK]
                    )
                    rt = jp * 32 + r
                    packbuf[slot, jp * PACK + (r >> 3), r & 7,
                            pl.ds(cc * LANES, LANES)] = val
                    del rt
            return c

        lax.fori_loop(0, JMAC, asm, 0)

        by_par(slot,
               lambda: fire_out(mg, slot, so0),
               lambda: fire_out(mg, slot, so1))
        return carry

    lax.fori_loop(0, MTOT_A, body, 0)
    drain_out(MTOT_A - 2, (MTOT_A - 2) & 1, so0)
    drain_out(MTOT_A - 1, (MTOT_A - 1) & 1, so1)


@functools.partial(
    pl.kernel,
    out_type=jax.ShapeDtypeStruct((N_FIELDS, WIDTH, BATCH), jnp.float32),
    mesh=_mesh,
    scratch_types=[
        pltpu.VMEM((BPW,), jnp.int32),
        pltpu.VMEM((2, BPW), jnp.int32),
        pltpu.VMEM((2, BPW), jnp.int32),
        pltpu.VMEM((2, BLK // 8, 8, 128), jnp.float32),
        pltpu.VMEM((2, PACK, 8, BLK), jnp.float32),
        pltpu.SemaphoreType.DMA,
        pltpu.SemaphoreType.DMA,
        pltpu.SemaphoreType.DMA,
        pltpu.SemaphoreType.DMA,
    ],
    compiler_params=pltpu.CompilerParams(needs_layout_passes=False),
)
def _embed_gather(xs_hbm, tab4_hbm, out_hbm, xsv, idx4, subv, rows, tiles,
                  sg0, sg1, so0, so1):
    wid = lax.axis_index("s") * NC + lax.axis_index("c")
    base_b = wid * BPW
    lane = lax.iota(jnp.int32, LANES)

    def prep_field(f):
        pltpu.sync_copy(xs_hbm.at[pl.ds(f * BATCH + base_b, BPW)], xsv)
        fslot = f & 1
        fbase = f * ROWS4_F

        def go(i, c):
            sl = pl.ds(i * LANES, LANES)
            v = xsv[sl]
            idx4[fslot, sl] = fbase + (v >> 2)
            subv[fslot, sl] = v & 3
            return c

        lax.fori_loop(0, BPW // LANES, go, 0)

    def fire_g(mg, slot, sem):
        f = mg >> 2
        blk = mg & 3
        fslot = f & 1

        def go(g, c):
            pltpu.async_copy(
                tab4_hbm.at[idx4.at[fslot, pl.ds(blk * BLK + g * 8, 8)]],
                rows.at[slot, g],
                sem,
            )
            return c

        lax.fori_loop(0, BLK // 8, go, 0)

    def drain_g(mg, slot, sem):
        f = mg >> 2
        blk = mg & 3
        fslot = f & 1

        def go(g, c):
            pltpu.make_async_copy(
                tab4_hbm.at[idx4.at[fslot, pl.ds(blk * BLK + g * 8, 8)]],
                rows.at[slot, g],
                sem,
            ).wait()
            return c

        lax.fori_loop(0, BLK // 8, go, 0)

    def fire_out(mg, slot, sem):
        f = mg >> 2
        blk = mg & 3
        b0 = base_b + blk * BLK

        def go(w8, c):
            pltpu.async_copy(
                tiles.at[slot, w8],
                out_hbm.at[f, pl.ds(w8 * 8, 8), pl.ds(b0, BLK)],
                sem,
            )
            return c

        lax.fori_loop(0, PACK, go, 0)

    def drain_out(mg, slot, sem):
        f = mg >> 2
        blk = mg & 3
        b0 = base_b + blk * BLK

        def go(w8, c):
            pltpu.make_async_copy(
                tiles.at[slot, w8],
                out_hbm.at[f, pl.ds(w8 * 8, 8), pl.ds(b0, BLK)],
                sem,
            ).wait()
            return c

        lax.fori_loop(0, PACK, go, 0)

    def by_par(slot, fn0, fn1):
        @pl.when(slot == 0)
        def _():
            fn0()

        @pl.when(slot != 0)
        def _():
            fn1()

    prep_field(0)
    fire_g(0, 0, sg0)

    def body(mg, carry):
        slot = mg & 1
        blk = mg & 3
        sslot = lane * 0 + slot

        @pl.when(mg + 1 < MTOT_B)
        def _():
            @pl.when(blk == 3)
            def _():
                prep_field((mg >> 2) + 1)

            by_par(1 - slot,
                   lambda: fire_g(mg + 1, 1 - slot, sg0),
                   lambda: fire_g(mg + 1, 1 - slot, sg1))

        by_par(slot,
               lambda: drain_g(mg, slot, sg0),
               lambda: drain_g(mg, slot, sg1))

        @pl.when(mg >= 2)
        def _():
            by_par(slot,
                   lambda: drain_out(mg - 2, slot, so0),
                   lambda: drain_out(mg - 2, slot, so1))

        fslot = (mg >> 2) & 1

        def asm(kk, c):
            b16 = kk * LANES + lane
            sub16 = subv[fslot, pl.ds(blk * BLK + kk * LANES, LANES)]
            for w in range(WIDTH):
                val = plsc.load_gather(
                    rows, [sslot, b16 >> 3, b16 & 7, sub16 + w * PACK]
                )
                tiles[slot, w // 8, w % 8, pl.ds(kk * LANES, LANES)] = val
            return c

        lax.fori_loop(0, BLK // LANES, asm, 0)

        by_par(slot,
               lambda: fire_out(mg, slot, so0),
               lambda: fire_out(mg, slot, so1))
        return carry

    lax.fori_loop(0, MTOT_B, body, 0)
    drain_out(MTOT_B - 2, (MTOT_B - 2) & 1, so0)
    drain_out(MTOT_B - 1, (MTOT_B - 1) & 1, so1)


def kernel(xs, tables):
    xs_flat = xs.reshape(N_FIELDS * BATCH)
    tab_t = jnp.transpose(tables, (0, 2, 1))
    tab4 = _detile(tab_t)
    out_t = _embed_gather(xs_flat, tab4)
    return jnp.transpose(out_t, (0, 2, 1))


# speedup vs baseline: 2.7296x; 1.0008x over previous
import functools

import jax
import jax.numpy as jnp
from jax import lax
from jax.experimental import pallas as pl
from jax.experimental.pallas import tpu as pltpu
from jax.experimental.pallas import tpu_sc as plsc

N_FIELDS = 26
VOCAB = 100000
WIDTH = 32
BATCH = 16384

PACK = 4
NTILE = 782
ROWS4_F = NTILE * 32
ROWS4 = N_FIELDS * ROWS4_F

NC = 2
NS = 16
NW = NC * NS
JPW = 25
JMAC = 5
MPF = JPW // JMAC
MTOT_A = N_FIELDS * MPF

BPW = BATCH // NW
BLK = 128
NBLK = BPW // BLK
MTOT_B = N_FIELDS * NBLK
LANES = 16

_mesh = plsc.VectorSubcoreMesh(core_axis_name="c", subcore_axis_name="s")


@functools.partial(
    pl.kernel,
    out_type=jax.ShapeDtypeStruct((ROWS4, 128), jnp.float32),
    mesh=_mesh,
    scratch_types=[
        pltpu.VMEM((2, PACK, JMAC, 8, 128), jnp.float32),
        pltpu.VMEM((2, PACK * JMAC, 8, 128), jnp.float32),
        pltpu.SemaphoreType.DMA,
        pltpu.SemaphoreType.DMA,
        pltpu.SemaphoreType.DMA,
        pltpu.SemaphoreType.DMA,
    ],
    compiler_params=pltpu.CompilerParams(needs_layout_passes=False),
)
def _detile(tab_hbm, out_hbm, stage, packbuf, si0, si1, so0, so1):
    wid = lax.axis_index("s") * NC + lax.axis_index("c")
    j0 = lax.min(wid * JPW, NTILE - JPW)
    lane = lax.iota(jnp.int32, LANES)

    i0s, i1s, quads = [], [], []
    for cc in range(8):
        c16 = cc * LANES + lane
        i0s.append(c16 >> 5)
        i1s.append((c16 >> 2) & 7)
        quads.append(c16 & 3)

    def step_fm(mg):
        return mg // MPF, mg % MPF

    def fire_in(mg, slot, sem):
        f, m = step_fm(mg)
        jbase = (j0 + m * JMAC) * 128

        def go(q, c):
            w8 = q & 3
            jp = q >> 2
            pltpu.async_copy(
                tab_hbm.at[f, pl.ds(w8 * 8, 8), pl.ds(jbase + jp * 128, 128)],
                stage.at[slot, w8, jp],
                sem,
            )
            return c

        lax.fori_loop(0, PACK * JMAC, go, 0)

    def drain_in(mg, slot, sem):
        f, m = step_fm(mg)
        jbase = (j0 + m * JMAC) * 128

        def go(q, c):
            w8 = q & 3
            jp = q >> 2
            pltpu.make_async_copy(
                tab_hbm.at[f, pl.ds(w8 * 8, 8), pl.ds(jbase + jp * 128, 128)],
                stage.at[slot, w8, jp],
                sem,
            ).wait()
            return c

        lax.fori_loop(0, PACK * JMAC, go, 0)

    def fire_out(mg, slot, sem):
        f, m = step_fm(mg)
        r0 = f * ROWS4_F + (j0 + m * JMAC) * 32

        def go(q, c):
            pltpu.async_copy(
                packbuf.at[slot, q],
                out_hbm.at[pl.ds(r0 + q * 8, 8), :],
                sem,
            )
            return c

        lax.fori_loop(0, PACK * JMAC, go, 0)

    def drain_out(mg, slot, sem):
        f, m = step_fm(mg)
        r0 = f * ROWS4_F + (j0 + m * JMAC) * 32

        def go(q, c):
            pltpu.make_async_copy(
                packbuf.at[slot, q],
                out_hbm.at[pl.ds(r0 + q * 8, 8), :],
                sem,
            ).wait()
            return c

        lax.fori_loop(0, PACK * JMAC, go, 0)

    def by_par(slot, fn0, fn1):
        @pl.when(slot == 0)
        def _():
            fn0()

        @pl.when(slot != 0)
        def _():
            fn1()

    fire_in(0, 0, si0)

    def body(mg, carry):
        slot = mg & 1
        sslot = lane * 0 + slot

        @pl.when(mg + 1 < MTOT_A)
        def _():
            by_par(1 - slot,
                   lambda: fire_in(mg + 1, 1 - slot, si0),
                   lambda: fire_in(mg + 1, 1 - slot, si1))

        by_par(slot,
               lambda: drain_in(mg, slot, si0),
               lambda: drain_in(mg, slot, si1))

        @pl.when(mg >= 2)
        def _():
            by_par(slot,
                   lambda: drain_out(mg - 2, slot, so0),
                   lambda: drain_out(mg - 2, slot, so1))

        def asm(jp, c):
            sjp = lane * 0 + jp
            for cc in range(8):
                for r in range(32):
                    val = plsc.load_gather(
                        stage, [sslot, i0s[cc], sjp, i1s[cc], quads[cc] + r * PACK]
                    )
                    packbuf[slot, jp * PACK + (r >> 3), r & 7,
                            pl.ds(cc * LANES, LANES)] = val
            return c

        lax.fori_loop(0, JMAC, asm, 0)

        by_par(slot,
               lambda: fire_out(mg, slot, so0),
               lambda: fire_out(mg, slot, so1))
        return carry

    lax.fori_loop(0, MTOT_A, body, 0)
    drain_out(MTOT_A - 2, (MTOT_A - 2) & 1, so0)
    drain_out(MTOT_A - 1, (MTOT_A - 1) & 1, so1)


@functools.partial(
    pl.kernel,
    out_type=jax.ShapeDtypeStruct((N_FIELDS, WIDTH, BATCH), jnp.float32),
    mesh=_mesh,
    scratch_types=[
        pltpu.VMEM((BPW,), jnp.int32),
        pltpu.VMEM((2, BPW), jnp.int32),
        pltpu.VMEM((2, BPW), jnp.int32),
        pltpu.VMEM((2, BLK // 8, 8, 128), jnp.float32),
        pltpu.VMEM((2, PACK, 8, BLK), jnp.float32),
        pltpu.SemaphoreType.DMA,
        pltpu.SemaphoreType.DMA,
        pltpu.SemaphoreType.DMA,
        pltpu.SemaphoreType.DMA,
    ],
    compiler_params=pltpu.CompilerParams(needs_layout_passes=False),
)
def _embed_gather(xs_hbm, tab4_hbm, out_hbm, xsv, idx4, subv, rows, tiles,
                  sg0, sg1, so0, so1):
    wid = lax.axis_index("s") * NC + lax.axis_index("c")
    base_b = wid * BPW
    lane = lax.iota(jnp.int32, LANES)

    def prep_field(f):
        pltpu.sync_copy(xs_hbm.at[pl.ds(f * BATCH + base_b, BPW)], xsv)
        fslot = f & 1
        fbase = f * ROWS4_F

        def go(i, c):
            sl = pl.ds(i * LANES, LANES)
            v = xsv[sl]
            idx4[fslot, sl] = fbase + (v >> 2)
            subv[fslot, sl] = v & 3
            return c

        lax.fori_loop(0, BPW // LANES, go, 0)

    def fire_g(mg, slot, sem):
        f = mg >> 2
        blk = mg & 3
        fslot = f & 1

        def go(g, c):
            pltpu.async_copy(
                tab4_hbm.at[idx4.at[fslot, pl.ds(blk * BLK + g * 8, 8)]],
                rows.at[slot, g],
                sem,
            )
            return c

        lax.fori_loop(0, BLK // 8, go, 0)

    def drain_g(mg, slot, sem):
        f = mg >> 2
        blk = mg & 3
        fslot = f & 1

        def go(g, c):
            pltpu.make_async_copy(
                tab4_hbm.at[idx4.at[fslot, pl.ds(blk * BLK + g * 8, 8)]],
                rows.at[slot, g],
                sem,
            ).wait()
            return c

        lax.fori_loop(0, BLK // 8, go, 0)

    def fire_out(mg, slot, sem):
        f = mg >> 2
        blk = mg & 3
        b0 = base_b + blk * BLK

        def go(w8, c):
            pltpu.async_copy(
                tiles.at[slot, w8],
                out_hbm.at[f, pl.ds(w8 * 8, 8), pl.ds(b0, BLK)],
                sem,
            )
            return c

        lax.fori_loop(0, PACK, go, 0)

    def drain_out(mg, slot, sem):
        f = mg >> 2
        blk = mg & 3
        b0 = base_b + blk * BLK

        def go(w8, c):
            pltpu.make_async_copy(
                tiles.at[slot, w8],
                out_hbm.at[f, pl.ds(w8 * 8, 8), pl.ds(b0, BLK)],
                sem,
            ).wait()
            return c

        lax.fori_loop(0, PACK, go, 0)

    def by_par(slot, fn0, fn1):
        @pl.when(slot == 0)
        def _():
            fn0()

        @pl.when(slot != 0)
        def _():
            fn1()

    prep_field(0)
    fire_g(0, 0, sg0)

    def body(mg, carry):
        slot = mg & 1
        blk = mg & 3
        sslot = lane * 0 + slot

        @pl.when(mg + 1 < MTOT_B)
        def _():
            @pl.when(blk == 3)
            def _():
                prep_field((mg >> 2) + 1)

            by_par(1 - slot,
                   lambda: fire_g(mg + 1, 1 - slot, sg0),
                   lambda: fire_g(mg + 1, 1 - slot, sg1))

        by_par(slot,
               lambda: drain_g(mg, slot, sg0),
               lambda: drain_g(mg, slot, sg1))

        @pl.when(mg >= 2)
        def _():
            by_par(slot,
                   lambda: drain_out(mg - 2, slot, so0),
                   lambda: drain_out(mg - 2, slot, so1))

        fslot = (mg >> 2) & 1

        def asm(kk, c):
            b16 = kk * LANES + lane
            sub16 = subv[fslot, pl.ds(blk * BLK + kk * LANES, LANES)]
            for w in range(WIDTH):
                val = plsc.load_gather(
                    rows, [sslot, b16 >> 3, b16 & 7, sub16 + w * PACK]
                )
                tiles[slot, w // 8, w % 8, pl.ds(kk * LANES, LANES)] = val
            return c

        lax.fori_loop(0, BLK // LANES, asm, 0)

        by_par(slot,
               lambda: fire_out(mg, slot, so0),
               lambda: fire_out(mg, slot, so1))
        return carry

    lax.fori_loop(0, MTOT_B, body, 0)
    drain_out(MTOT_B - 2, (MTOT_B - 2) & 1, so0)
    drain_out(MTOT_B - 1, (MTOT_B - 1) & 1, so1)


def kernel(xs, tables):
    xs_flat = xs.reshape(N_FIELDS * BATCH)
    tab_t = jnp.transpose(tables, (0, 2, 1))
    tab4 = _detile(tab_t)
    out_t = _embed_gather(xs_flat, tab4)
    return jnp.transpose(out_t, (0, 2, 1))


# speedup vs baseline: 2.7634x; 1.0124x over previous
import functools

import jax
import jax.numpy as jnp
from jax import lax
from jax.experimental import pallas as pl
from jax.experimental.pallas import tpu as pltpu
from jax.experimental.pallas import tpu_sc as plsc

N_FIELDS = 26
VOCAB = 100000
WIDTH = 32
BATCH = 16384

PACK = 4
NTILE = 782
ROWS4_F = NTILE * 32
ROWS4 = N_FIELDS * ROWS4_F

NC = 2
NS = 16
NW = NC * NS
JPW = 25
JMAC = 5
MPF = JPW // JMAC
MTOT_A = N_FIELDS * MPF

BPW = BATCH // NW
BLK = 128
NBLK = BPW // BLK
MTOT_B = N_FIELDS * NBLK
LANES = 16

_mesh = plsc.VectorSubcoreMesh(core_axis_name="c", subcore_axis_name="s")


@functools.partial(
    pl.kernel,
    out_type=jax.ShapeDtypeStruct((ROWS4, 128), jnp.float32),
    mesh=_mesh,
    scratch_types=[
        pltpu.VMEM((2, PACK, JMAC, 8, 128), jnp.float32),
        pltpu.VMEM((2, PACK * JMAC, 8, 128), jnp.float32),
        pltpu.SemaphoreType.DMA,
        pltpu.SemaphoreType.DMA,
        pltpu.SemaphoreType.DMA,
        pltpu.SemaphoreType.DMA,
    ],
    compiler_params=pltpu.CompilerParams(needs_layout_passes=False),
)
def _detile(tab_hbm, out_hbm, stage, packbuf, si0, si1, so0, so1):
    wid = lax.axis_index("s") * NC + lax.axis_index("c")
    j0 = lax.min(wid * JPW, NTILE - JPW)
    lane = lax.iota(jnp.int32, LANES)

    i0s, i1s, quads = [], [], []
    for cc in range(8):
        c16 = cc * LANES + lane
        i0s.append(c16 >> 5)
        i1s.append((c16 >> 2) & 7)
        quads.append(c16 & 3)

    def step_fm(mg):
        return mg // MPF, mg % MPF

    def fire_in(mg, slot, sem):
        f, m = step_fm(mg)
        jbase = (j0 + m * JMAC) * 128

        def go(q, c):
            w8 = q & 3
            jp = q >> 2
            pltpu.async_copy(
                tab_hbm.at[f, pl.ds(w8 * 8, 8), pl.ds(jbase + jp * 128, 128)],
                stage.at[slot, w8, jp],
                sem,
            )
            return c

        lax.fori_loop(0, PACK * JMAC, go, 0, unroll=4)

    def drain_in(mg, slot, sem):
        f, m = step_fm(mg)
        jbase = (j0 + m * JMAC) * 128

        def go(q, c):
            w8 = q & 3
            jp = q >> 2
            pltpu.make_async_copy(
                tab_hbm.at[f, pl.ds(w8 * 8, 8), pl.ds(jbase + jp * 128, 128)],
                stage.at[slot, w8, jp],
                sem,
            ).wait()
            return c

        lax.fori_loop(0, PACK * JMAC, go, 0, unroll=4)

    def fire_out(mg, slot, sem):
        f, m = step_fm(mg)
        r0 = f * ROWS4_F + (j0 + m * JMAC) * 32

        def go(q, c):
            pltpu.async_copy(
                packbuf.at[slot, q],
                out_hbm.at[pl.ds(r0 + q * 8, 8), :],
                sem,
            )
            return c

        lax.fori_loop(0, PACK * JMAC, go, 0, unroll=4)

    def drain_out(mg, slot, sem):
        f, m = step_fm(mg)
        r0 = f * ROWS4_F + (j0 + m * JMAC) * 32

        def go(q, c):
            pltpu.make_async_copy(
                packbuf.at[slot, q],
                out_hbm.at[pl.ds(r0 + q * 8, 8), :],
                sem,
            ).wait()
            return c

        lax.fori_loop(0, PACK * JMAC, go, 0, unroll=4)

    def by_par(slot, fn0, fn1):
        @pl.when(slot == 0)
        def _():
            fn0()

        @pl.when(slot != 0)
        def _():
            fn1()

    fire_in(0, 0, si0)

    def body(mg, carry):
        slot = mg & 1
        sslot = lane * 0 + slot

        @pl.when(mg + 1 < MTOT_A)
        def _():
            by_par(1 - slot,
                   lambda: fire_in(mg + 1, 1 - slot, si0),
                   lambda: fire_in(mg + 1, 1 - slot, si1))

        by_par(slot,
               lambda: drain_in(mg, slot, si0),
               lambda: drain_in(mg, slot, si1))

        @pl.when(mg >= 2)
        def _():
            by_par(slot,
                   lambda: drain_out(mg - 2, slot, so0),
                   lambda: drain_out(mg - 2, slot, so1))

        def asm(jp, c):
            sjp = lane * 0 + jp
            for cc in range(8):
                for r in range(32):
                    val = plsc.load_gather(
                        stage, [sslot, i0s[cc], sjp, i1s[cc], quads[cc] + r * PACK]
                    )
                    packbuf[slot, jp * PACK + (r >> 3), r & 7,
                            pl.ds(cc * LANES, LANES)] = val
            return c

        lax.fori_loop(0, JMAC, asm, 0)

        by_par(slot,
               lambda: fire_out(mg, slot, so0),
               lambda: fire_out(mg, slot, so1))
        return carry

    lax.fori_loop(0, MTOT_A, body, 0)
    drain_out(MTOT_A - 2, (MTOT_A - 2) & 1, so0)
    drain_out(MTOT_A - 1, (MTOT_A - 1) & 1, so1)


@functools.partial(
    pl.kernel,
    out_type=jax.ShapeDtypeStruct((N_FIELDS, WIDTH, BATCH), jnp.float32),
    mesh=_mesh,
    scratch_types=[
        pltpu.VMEM((BPW,), jnp.int32),
        pltpu.VMEM((2, BPW), jnp.int32),
        pltpu.VMEM((2, BPW), jnp.int32),
        pltpu.VMEM((2, BLK // 8, 8, 128), jnp.float32),
        pltpu.VMEM((2, PACK, 8, BLK), jnp.float32),
        pltpu.SemaphoreType.DMA,
        pltpu.SemaphoreType.DMA,
        pltpu.SemaphoreType.DMA,
        pltpu.SemaphoreType.DMA,
    ],
    compiler_params=pltpu.CompilerParams(needs_layout_passes=False),
)
def _embed_gather(xs_hbm, tab4_hbm, out_hbm, xsv, idx4, subv, rows, tiles,
                  sg0, sg1, so0, so1):
    wid = lax.axis_index("s") * NC + lax.axis_index("c")
    base_b = wid * BPW
    lane = lax.iota(jnp.int32, LANES)

    def prep_field(f):
        pltpu.sync_copy(xs_hbm.at[pl.ds(f * BATCH + base_b, BPW)], xsv)
        fslot = f & 1
        fbase = f * ROWS4_F

        def go(i, c):
            sl = pl.ds(i * LANES, LANES)
            v = xsv[sl]
            idx4[fslot, sl] = fbase + (v >> 2)
            subv[fslot, sl] = v & 3
            return c

        lax.fori_loop(0, BPW // LANES, go, 0)

    def fire_g(mg, slot, sem):
        f = mg >> 2
        blk = mg & 3
        fslot = f & 1

        def go(g, c):
            pltpu.async_copy(
                tab4_hbm.at[idx4.at[fslot, pl.ds(blk * BLK + g * 8, 8)]],
                rows.at[slot, g],
                sem,
            )
            return c

        lax.fori_loop(0, BLK // 8, go, 0, unroll=4)

    def drain_g(mg, slot, sem):
        f = mg >> 2
        blk = mg & 3
        fslot = f & 1

        def go(g, c):
            pltpu.make_async_copy(
                tab4_hbm.at[idx4.at[fslot, pl.ds(blk * BLK + g * 8, 8)]],
                rows.at[slot, g],
                sem,
            ).wait()
            return c

        lax.fori_loop(0, BLK // 8, go, 0, unroll=4)

    def fire_out(mg, slot, sem):
        f = mg >> 2
        blk = mg & 3
        b0 = base_b + blk * BLK

        def go(w8, c):
            pltpu.async_copy(
                tiles.at[slot, w8],
                out_hbm.at[f, pl.ds(w8 * 8, 8), pl.ds(b0, BLK)],
                sem,
            )
            return c

        lax.fori_loop(0, PACK, go, 0)

    def drain_out(mg, slot, sem):
        f = mg >> 2
        blk = mg & 3
        b0 = base_b + blk * BLK

        def go(w8, c):
            pltpu.make_async_copy(
                tiles.at[slot, w8],
                out_hbm.at[f, pl.ds(w8 * 8, 8), pl.ds(b0, BLK)],
                sem,
            ).wait()
            return c

        lax.fori_loop(0, PACK, go, 0)

    def by_par(slot, fn0, fn1):
        @pl.when(slot == 0)
        def _():
            fn0()

        @pl.when(slot != 0)
        def _():
            fn1()

    prep_field(0)
    fire_g(0, 0, sg0)

    def body(mg, carry):
        slot = mg & 1
        blk = mg & 3
        sslot = lane * 0 + slot

        @pl.when(mg + 1 < MTOT_B)
        def _():
            @pl.when(blk == 3)
            def _():
                prep_field((mg >> 2) + 1)

            by_par(1 - slot,
                   lambda: fire_g(mg + 1, 1 - slot, sg0),
                   lambda: fire_g(mg + 1, 1 - slot, sg1))

        by_par(slot,
               lambda: drain_g(mg, slot, sg0),
               lambda: drain_g(mg, slot, sg1))

        @pl.when(mg >= 2)
        def _():
            by_par(slot,
                   lambda: drain_out(mg - 2, slot, so0),
                   lambda: drain_out(mg - 2, slot, so1))

        fslot = (mg >> 2) & 1

        def asm(kk, c):
            b16 = kk * LANES + lane
            sub16 = subv[fslot, pl.ds(blk * BLK + kk * LANES, LANES)]
            for w in range(WIDTH):
                val = plsc.load_gather(
                    rows, [sslot, b16 >> 3, b16 & 7, sub16 + w * PACK]
                )
                tiles[slot, w // 8, w % 8, pl.ds(kk * LANES, LANES)] = val
            return c

        lax.fori_loop(0, BLK // LANES, asm, 0)

        by_par(slot,
               lambda: fire_out(mg, slot, so0),
               lambda: fire_out(mg, slot, so1))
        return carry

    lax.fori_loop(0, MTOT_B, body, 0)
    drain_out(MTOT_B - 2, (MTOT_B - 2) & 1, so0)
    drain_out(MTOT_B - 1, (MTOT_B - 1) & 1, so1)


def kernel(xs, tables):
    xs_flat = xs.reshape(N_FIELDS * BATCH)
    tab_t = jnp.transpose(tables, (0, 2, 1))
    tab4 = _detile(tab_t)
    out_t = _embed_gather(xs_flat, tab4)
    return jnp.transpose(out_t, (0, 2, 1))
